# Initial kernel scaffold; baseline (speedup 1.0000x reference)
import jax
import jax.numpy as jnp
from jax.experimental import pallas as pl


def kernel(index, length, data):
    raise NotImplementedError("write your pallas kernel here")


# speedup vs baseline: 3.6347x; 3.6347x over previous
import functools

import jax
import jax.numpy as jnp
from jax import lax
from jax.experimental import pallas as pl
from jax.experimental.pallas import tpu as pltpu
from jax.experimental.pallas import tpu_sc as plsc

CYCLE = 168
L_OUT = 200
CH = 64
PACK = 8
M_PER_B = L_OUT // PACK
ROW_W = PACK * CH
NC = 2
NS = 16
NW = NC * NS
ROWS_PER_DMA = 128


def _sc_window_gather(base_idx, data8):
    B = base_idx.shape[0]
    b_per_w = B // NW
    rows_w = b_per_w * M_PER_B
    n_chunks = rows_w // ROWS_PER_DMA

    mesh = plsc.VectorSubcoreMesh(core_axis_name="c", subcore_axis_name="s")

    @functools.partial(
        pl.kernel,
        out_type=jax.ShapeDtypeStruct((B * M_PER_B, ROW_W), jnp.float32),
        mesh=mesh,
        compiler_params=pltpu.CompilerParams(needs_layout_passes=False),
        scratch_types=[
            pltpu.VMEM((b_per_w,), jnp.int32),
            pltpu.VMEM((rows_w,), jnp.int32),
            pltpu.VMEM((ROWS_PER_DMA, ROW_W), jnp.float32),
            pltpu.SemaphoreType.DMA,
            pltpu.SemaphoreType.DMA,
        ],
    )
    def k(idx_hbm, data_hbm, out_hbm, idx_v, idx_buf, buf, gsem, ssem):
        wid = lax.axis_index("s") * NC + lax.axis_index("c")
        b0 = wid * b_per_w
        pltpu.sync_copy(idx_hbm.at[pl.ds(b0, b_per_w)], idx_v)
        lane = lax.broadcasted_iota(jnp.int32, (16,), 0)
        for g in range(b_per_w // 16):
            v = idx_v[pl.ds(g * 16, 16)]
            offs0 = (g * 16 + lane) * M_PER_B
            for m in range(M_PER_B):
                plsc.store_scatter(idx_buf, [offs0 + m], v)
                v = v + PACK
                v = jnp.where(v >= CYCLE, v - CYCLE, v)

        row0 = wid * rows_w

        def chunk(c, carry):
            isl = idx_buf.at[pl.ds(c * ROWS_PER_DMA, ROWS_PER_DMA)]
            pltpu.async_copy(data_hbm.at[isl], buf, gsem).wait()
            pltpu.async_copy(
                buf, out_hbm.at[pl.ds(row0 + c * ROWS_PER_DMA, ROWS_PER_DMA)],
                ssem).wait()
            return carry

        lax.fori_loop(0, n_chunks, chunk, 0)

    return k(base_idx, data8)


def kernel(index, length, data):
    B = index.shape[0]
    base_idx = ((index.reshape(B).astype(jnp.int32) + (length - L_OUT))
                % CYCLE).astype(jnp.int32)
    data8 = jnp.concatenate(
        [jnp.roll(data, -r, axis=0) for r in range(PACK)], axis=1)
    out = _sc_window_gather(base_idx, data8)
    return out.reshape(B, L_OUT, CH)


# speedup vs baseline: 3.6503x; 1.0043x over previous
import functools

import jax
import jax.numpy as jnp
from jax import lax
from jax.experimental import pallas as pl
from jax.experimental.pallas import tpu as pltpu
from jax.experimental.pallas import tpu_sc as plsc

CYCLE = 168
L_OUT = 200
CH = 64
PACK = 8
M_PER_B = L_OUT // PACK
ROW_W = PACK * CH
NC = 2
NS = 16
NW = NC * NS
CHUNK = 64


def _sc_window_gather(base_idx, data8):
    B = base_idx.shape[0]
    b_per_w = B // NW
    rows_w = b_per_w * M_PER_B
    n_chunks = rows_w // CHUNK

    mesh = plsc.VectorSubcoreMesh(core_axis_name="c", subcore_axis_name="s")

    @functools.partial(
        pl.kernel,
        out_type=jax.ShapeDtypeStruct((B * M_PER_B, ROW_W), jnp.float32),
        mesh=mesh,
        compiler_params=pltpu.CompilerParams(needs_layout_passes=False),
        scratch_types=[
            pltpu.VMEM((b_per_w,), jnp.int32),
            pltpu.VMEM((rows_w,), jnp.int32),
            pltpu.VMEM((CHUNK, ROW_W), jnp.float32),
            pltpu.VMEM((CHUNK, ROW_W), jnp.float32),
            pltpu.SemaphoreType.DMA,
            pltpu.SemaphoreType.DMA,
            pltpu.SemaphoreType.DMA,
        ],
    )
    def k(idx_hbm, data_hbm, out_hbm, idx_v, idx_buf, buf_a, buf_b,
          gsem, ssem_a, ssem_b):
        wid = lax.axis_index("s") * NC + lax.axis_index("c")
        b0 = wid * b_per_w
        pltpu.sync_copy(idx_hbm.at[pl.ds(b0, b_per_w)], idx_v)
        lane = lax.broadcasted_iota(jnp.int32, (16,), 0)
        for g in range(b_per_w // 16):
            v = idx_v[pl.ds(g * 16, 16)]
            offs0 = (g * 16 + lane) * M_PER_B
            for m in range(M_PER_B):
                plsc.store_scatter(idx_buf, [offs0 + m], v)
                v = v + PACK
                v = jnp.where(v >= CYCLE, v - CYCLE, v)

        row0 = wid * rows_w
        bufs = (buf_a, buf_b)
        ssems = (ssem_a, ssem_b)

        def out_slice(c):
            return out_hbm.at[pl.ds(row0 + c * CHUNK, CHUNK)]

        def pair(q, carry):
            for b in (0, 1):
                c = 2 * q + b
                buf, ssem = bufs[b], ssems[b]

                @pl.when(q >= 1)
                def _wait_prev(buf=buf, ssem=ssem, c=c):
                    pltpu.make_async_copy(buf, out_slice(c - 2), ssem).wait()

                isl = idx_buf.at[pl.ds(c * CHUNK, CHUNK)]
                pltpu.async_copy(data_hbm.at[isl], buf, gsem).wait()
                pltpu.async_copy(buf, out_slice(c), ssem)
            return carry

        lax.fori_loop(0, n_chunks // 2, pair, 0)
        for b in (0, 1):
            c = n_chunks - 2 + b
            pltpu.make_async_copy(bufs[b], out_slice(c), ssems[b]).wait()

    return k(base_idx, data8)


def kernel(index, length, data):
    B = index.shape[0]
    base_idx = ((index.reshape(B).astype(jnp.int32) + (length - L_OUT))
                % CYCLE).astype(jnp.int32)
    data8 = jnp.concatenate(
        [jnp.roll(data, -r, axis=0) for r in range(PACK)], axis=1)
    out = _sc_window_gather(base_idx, data8)
    return out.reshape(B, L_OUT, CH)
